# Initial kernel scaffold; baseline (speedup 1.0000x reference)
import jax
import jax.numpy as jnp
from jax.experimental import pallas as pl


def kernel(x, edge_index, Wq1, bq1, Wk1, bk1, Wv1, bv1, Ws1, bs1, Wq2, bq2, Wk2, bk2, Wv2, bv2, Ws2, bs2):
    raise NotImplementedError("write your pallas kernel here")


# speedup vs baseline: 5.6148x; 5.6148x over previous
import functools

import numpy as np
import jax
import jax.numpy as jnp
from jax import lax
from jax.experimental import pallas as pl
from jax.experimental.pallas import tpu as pltpu
from jax.experimental.pallas import tpu_sc as plsc

NC = 2
NS = 16
NW = NC * NS
CHUNK = 64
LANES = 16
RBLK = 1024


def _qkvs_stage(xp, Wq, Wk, Wv, Ws, bstack, brow):
    npad, d = xp.shape
    grid = (npad // RBLK,)

    def body(x_ref, wq_ref, wk_ref, wv_ref, ws_ref, b_ref, q_out, kv_out, s_out):
        xb = x_ref[...]
        b = b_ref[...]
        q_out[...] = jnp.dot(xb, wq_ref[...], preferred_element_type=jnp.float32) + b[brow][None, :]
        kv_out[:, :d] = jnp.dot(xb, wk_ref[...], preferred_element_type=jnp.float32) + b[brow + 1][None, :]
        kv_out[:, d:] = jnp.dot(xb, wv_ref[...], preferred_element_type=jnp.float32) + b[brow + 2][None, :]
        s_out[...] = jnp.dot(xb, ws_ref[...], preferred_element_type=jnp.float32) + b[brow + 3][None, :]

    wspec = pl.BlockSpec((d, d), lambda i: (0, 0))
    return pl.pallas_call(
        body,
        grid=grid,
        in_specs=[
            pl.BlockSpec((RBLK, d), lambda i: (i, 0)),
            wspec, wspec, wspec, wspec,
            pl.BlockSpec((8, d), lambda i: (0, 0)),
        ],
        out_specs=[
            pl.BlockSpec((RBLK, d), lambda i: (i, 0)),
            pl.BlockSpec((RBLK, 2 * d), lambda i: (i, 0)),
            pl.BlockSpec((RBLK, d), lambda i: (i, 0)),
        ],
        out_shape=[
            jax.ShapeDtypeStruct((npad, d), jnp.float32),
            jax.ShapeDtypeStruct((npad, 2 * d), jnp.float32),
            jax.ShapeDtypeStruct((npad, d), jnp.float32),
        ],
    )(xp, Wq, Wk, Wv, Ws, bstack)


def _combine(p_ref, d_ref, s_ref, d):
    p = p_ref[0] + p_ref[1]
    dsum = jnp.sum(d_ref[...], axis=(0, 1))
    rows = lax.broadcasted_iota(jnp.int32, (RBLK, d), 0)
    cols = lax.broadcasted_iota(jnp.int32, (RBLK, d), 1)
    sel = (lax.broadcasted_iota(jnp.int32, (RBLK, RBLK // d), 1)
           == lax.broadcasted_iota(jnp.int32, (RBLK, RBLK // d), 0) // d
           ).astype(jnp.float32)
    expanded = jnp.dot(sel, dsum, preferred_element_type=jnp.float32)
    picked = jnp.where(cols == rows % d, expanded, 0.0)
    denom = jnp.sum(picked, axis=1, keepdims=True) + 1e-16
    return p / denom + s_ref[...]


def _mid_stage(part, dpart, skip, Wq, Wk, Wv, Ws, bstack, brow):
    _, npad, d = part.shape
    grid = (npad // RBLK,)

    def body(p_ref, d_ref, s_ref, wq_ref, wk_ref, wv_ref, ws_ref, b_ref,
             q_out, kv_out, s_out):
        h = jnp.maximum(_combine(p_ref, d_ref, s_ref, d), 0.0)
        b = b_ref[...]
        q_out[...] = jnp.dot(h, wq_ref[...], preferred_element_type=jnp.float32) + b[brow][None, :]
        kv_out[:, :d] = jnp.dot(h, wk_ref[...], preferred_element_type=jnp.float32) + b[brow + 1][None, :]
        kv_out[:, d:] = jnp.dot(h, wv_ref[...], preferred_element_type=jnp.float32) + b[brow + 2][None, :]
        s_out[...] = jnp.dot(h, ws_ref[...], preferred_element_type=jnp.float32) + b[brow + 3][None, :]

    wspec = pl.BlockSpec((d, d), lambda i: (0, 0))
    return pl.pallas_call(
        body,
        grid=grid,
        in_specs=[
            pl.BlockSpec((2, RBLK, d), lambda i: (0, i, 0)),
            pl.BlockSpec((2, NS, RBLK // d, d), lambda i: (0, 0, i, 0)),
            pl.BlockSpec((RBLK, d), lambda i: (i, 0)),
            wspec, wspec, wspec, wspec,
            pl.BlockSpec((8, d), lambda i: (0, 0)),
        ],
        out_specs=[
            pl.BlockSpec((RBLK, d), lambda i: (i, 0)),
            pl.BlockSpec((RBLK, 2 * d), lambda i: (i, 0)),
            pl.BlockSpec((RBLK, d), lambda i: (i, 0)),
        ],
        out_shape=[
            jax.ShapeDtypeStruct((npad, d), jnp.float32),
            jax.ShapeDtypeStruct((npad, 2 * d), jnp.float32),
            jax.ShapeDtypeStruct((npad, d), jnp.float32),
        ],
    )(part, dpart, skip, Wq, Wk, Wv, Ws, bstack)


def _final_stage(part, dpart, skip):
    _, npad, d = part.shape
    grid = (npad // RBLK,)

    def body(p_ref, d_ref, s_ref, o_ref):
        o_ref[...] = _combine(p_ref, d_ref, s_ref, d)

    return pl.pallas_call(
        body,
        grid=grid,
        in_specs=[
            pl.BlockSpec((2, RBLK, d), lambda i: (0, i, 0)),
            pl.BlockSpec((2, NS, RBLK // d, d), lambda i: (0, 0, i, 0)),
            pl.BlockSpec((RBLK, d), lambda i: (i, 0)),
        ],
        out_specs=pl.BlockSpec((RBLK, d), lambda i: (i, 0)),
        out_shape=jax.ShapeDtypeStruct((npad, d), jnp.float32),
    )(part, dpart, skip)


def _edge_stage(q_tab, kv_tab, src_p, dst_p):
    npad, d = q_tab.shape
    epad = src_p.shape[0]
    epw = epad // NW
    kch = epw // CHUNK
    drows = npad // d
    rows_a = npad // NS
    scale = np.float32(1.0) / np.float32(np.sqrt(np.float32(d)))
    nvec = d // LANES
    assert rows_a % 8 == 0 and epw % CHUNK == 0 and npad % d == 0

    mesh = plsc.VectorSubcoreMesh(core_axis_name="c", subcore_axis_name="s")

    @functools.partial(
        pl.kernel,
        out_type=[
            jax.ShapeDtypeStruct((NC, npad, d), jnp.float32),
            jax.ShapeDtypeStruct((NC, NS, drows, d), jnp.float32),
        ],
        mesh=mesh,
        compiler_params=pltpu.CompilerParams(needs_layout_passes=False),
        scratch_types=[
            pltpu.VMEM((CHUNK,), jnp.int32),
            pltpu.VMEM((CHUNK,), jnp.int32),
            pltpu.VMEM((CHUNK, d), jnp.float32),
            pltpu.VMEM((CHUNK, 2 * d), jnp.float32),
            pltpu.VMEM((CHUNK, d), jnp.float32),
            pltpu.VMEM((CHUNK,), jnp.float32),
            pltpu.VMEM((8, d), jnp.float32),
            pltpu.VMEM((drows, d), jnp.float32),
            pltpu.VMEM_SHARED((npad, d), jnp.float32),
            pltpu.SemaphoreType.DMA,
            pltpu.SemaphoreType.DMA,
        ],
    )
    def edge_kernel(q_hbm, kv_hbm, src_hbm, dst_hbm, outv_hbm, outd_hbm,
                    sidx, didx, qbuf, kvbuf, sbuf, exbuf, zbuf, dden, acc,
                    sem1, sem2):
        cid = lax.axis_index("c")
        sid = lax.axis_index("s")
        wid = sid * NC + cid

        zero16 = jnp.zeros((LANES,), jnp.float32)
        for rr in range(8):
            for jj in range(d // LANES):
                zbuf[rr, pl.ds(jj * LANES, LANES)] = zero16
        def dzbody(r, _):
            for jj in range(d // LANES):
                dden[r, pl.ds(jj * LANES, LANES)] = zero16
            return 0

        lax.fori_loop(0, drows, dzbody, 0)

        zbase = sid * rows_a

        def zbody(i, _):
            pltpu.sync_copy(zbuf, acc.at[pl.ds(zbase + i * 8, 8)])
            return 0

        lax.fori_loop(0, rows_a // 8, zbody, 0)
        plsc.subcore_barrier()

        lane_iota = lax.iota(jnp.int32, LANES)
        lane_masks = [lane_iota == l for l in range(LANES)]

        def gbody(g, _):
            ebase = wid * epw + g * CHUNK
            pltpu.sync_copy(src_hbm.at[pl.ds(ebase, CHUNK)], sidx)
            pltpu.sync_copy(dst_hbm.at[pl.ds(ebase, CHUNK)], didx)
            cq = pltpu.async_copy(q_hbm.at[didx], qbuf, sem1)
            ckv = pltpu.async_copy(kv_hbm.at[sidx], kvbuf, sem2)
            cq.wait()
            ckv.wait()

            def ebody(e, _):
                parts = [qbuf[e, pl.ds(j * LANES, LANES)]
                         * kvbuf[e, pl.ds(j * LANES, LANES)]
                         for j in range(nvec)]
                while len(parts) > 1:
                    parts = [parts[i] + parts[i + 1]
                             for i in range(0, len(parts) - 1, 2)] \
                        + ([parts[-1]] if len(parts) % 2 else [])
                ev = jnp.exp(jnp.full((LANES,), jnp.sum(parts[0]) * scale,
                                      jnp.float32))
                plsc.store_scatter(exbuf, [jnp.full((LANES,), e, jnp.int32)],
                                   ev, mask=lane_masks[0])
                for j in range(nvec):
                    sbuf[e, pl.ds(j * LANES, LANES)] = \
                        kvbuf[e, pl.ds(d + j * LANES, LANES)] * ev
                return 0

            lax.fori_loop(0, CHUNK, ebody, 0)
            pltpu.sync_copy(sbuf, acc.at[didx], add=True)

            for s in range(CHUNK // LANES):
                dv = didx[pl.ds(s * LANES, LANES)]
                exv = exbuf[pl.ds(s * LANES, LANES)]
                row = lax.shift_right_logical(dv, 7)
                col = lax.bitwise_and(dv, jnp.full((LANES,), d - 1, jnp.int32))
                for l in range(LANES):
                    plsc.addupdate_scatter(dden, [row, col], exv,
                                           mask=lane_masks[l])
            return 0

        lax.fori_loop(0, kch, gbody, 0)
        plsc.subcore_barrier()

        obase = sid * rows_a
        pltpu.sync_copy(acc.at[pl.ds(obase, rows_a)],
                        outv_hbm.at[cid, pl.ds(obase, rows_a)])
        pltpu.sync_copy(dden, outd_hbm.at[cid, sid])

    return edge_kernel(q_tab, kv_tab, src_p, dst_p)


def kernel(x, edge_index, Wq1, bq1, Wk1, bk1, Wv1, bv1, Ws1, bs1,
           Wq2, bq2, Wk2, bk2, Wv2, bv2, Ws2, bs2):
    n, d = x.shape
    e = edge_index.shape[1]
    npad = ((n + RBLK - 1) // RBLK) * RBLK
    epad = ((e + NW * CHUNK - 1) // (NW * CHUNK)) * (NW * CHUNK)

    src = edge_index[0].astype(jnp.int32)
    dst = edge_index[1].astype(jnp.int32)
    fill = jnp.full((epad - e,), n, jnp.int32)
    src_p = jnp.concatenate([src, fill])
    dst_p = jnp.concatenate([dst, fill])
    xp = jnp.pad(x, ((0, npad - n), (0, 0)))
    bstack = jnp.stack([bq1, bk1, bv1, bs1, bq2, bk2, bv2, bs2])

    q1, kv1, skip1 = _qkvs_stage(xp, Wq1, Wk1, Wv1, Ws1, bstack, 0)
    part1, dpart1 = _edge_stage(q1, kv1, src_p, dst_p)
    q2, kv2, skip2 = _mid_stage(part1, dpart1, skip1, Wq2, Wk2, Wv2, Ws2,
                                bstack, 4)
    part2, dpart2 = _edge_stage(q2, kv2, src_p, dst_p)
    out = _final_stage(part2, dpart2, skip2)
    return out[:n]
